# Initial kernel scaffold; baseline (speedup 1.0000x reference)
import jax
import jax.numpy as jnp
from jax.experimental import pallas as pl


def kernel(x, pos_table):
    raise NotImplementedError("write your pallas kernel here")


# speedup vs baseline: 1.3827x; 1.3827x over previous
import jax
import jax.numpy as jnp
from jax.experimental import pallas as pl


def _add_body(x_ref, p_ref, o_ref):
    o_ref[...] = x_ref[...] + p_ref[...]


def kernel(x, pos_table):
    batch, seq_len, d_model = x.shape
    bs = 256
    grid = (batch, seq_len // bs)
    return pl.pallas_call(
        _add_body,
        grid=grid,
        in_specs=[
            pl.BlockSpec((1, bs, d_model), lambda b, s: (b, s, 0)),
            pl.BlockSpec((bs, d_model), lambda b, s: (s, 0)),
        ],
        out_specs=pl.BlockSpec((1, bs, d_model), lambda b, s: (b, s, 0)),
        out_shape=jax.ShapeDtypeStruct(x.shape, x.dtype),
    )(x, pos_table)
